# Initial kernel scaffold; baseline (speedup 1.0000x reference)
import functools

import jax
import jax.numpy as jnp
from jax.experimental import pallas as pl
from jax.experimental.pallas import tpu as pltpu

S = 2048
SS = S * S
SENT = 4 * SS

_ROWS = 8192
_BLK = 1024
_N_GRID = _ROWS // _BLK


def _transform_body(lat_ref, lon_ref, alt_ref, cam_ref, n_ref, idx_ref):
    lat = lat_ref[...]
    lon = lon_ref[...]
    alt = alt_ref[...]

    lat0 = cam_ref[0, 0]
    lon0 = cam_ref[0, 1]
    alt0 = cam_ref[0, 2]
    qs = cam_ref[0, 3]
    qx = cam_ref[0, 4]
    qy = cam_ref[0, 5]
    qz = cam_ref[0, 6]

    a = 6378137.0
    e2 = 6.69437999014e-3

    latr = jnp.deg2rad(lat)
    lonr = jnp.deg2rad(lon)
    s = jnp.sin(latr)
    c = jnp.cos(latr)
    Nv = a / jnp.sqrt(1.0 - e2 * s * s)
    X = (Nv + alt) * c * jnp.cos(lonr)
    Y = (Nv + alt) * c * jnp.sin(lonr)
    Z = (Nv * (1.0 - e2) + alt) * s

    latr0 = jnp.deg2rad(lat0)
    lonr0 = jnp.deg2rad(lon0)
    s0 = jnp.sin(latr0)
    c0 = jnp.cos(latr0)
    Nv0 = a / jnp.sqrt(1.0 - e2 * s0 * s0)
    x0 = (Nv0 + alt0) * c0 * jnp.cos(lonr0)
    y0 = (Nv0 + alt0) * c0 * jnp.sin(lonr0)
    z0 = (Nv0 * (1.0 - e2) + alt0) * s0

    dx, dy, dz = X - x0, Y - y0, Z - z0
    sl, cl = jnp.sin(latr0), jnp.cos(latr0)
    so, co = jnp.sin(lonr0), jnp.cos(lonr0)
    e = -so * dx + co * dy
    n = -sl * co * dx - sl * so * dy + cl * dz
    u = cl * co * dx + cl * so * dy + sl * dz

    nqs = -qs
    nrm = jnp.sqrt(nqs * nqs + qx * qx + qy * qy + qz * qz) + 1e-12
    rqs, rqx, rqy, rqz = nqs / nrm, qx / nrm, qy / nrm, qz / nrm
    x = ((1 - 2 * (rqy * rqy + rqz * rqz)) * e
         + 2 * (rqx * rqy - rqz * rqs) * n
         + 2 * (rqx * rqz + rqy * rqs) * u)
    y = (2 * (rqx * rqy + rqz * rqs) * e
         + (1 - 2 * (rqx * rqx + rqz * rqz)) * n
         + 2 * (rqy * rqz - rqx * rqs) * u)
    z = (2 * (rqx * rqz - rqy * rqs) * e
         + 2 * (rqy * rqz + rqx * rqs) * n
         + (1 - 2 * (rqx * rqx + rqy * rqy)) * u)

    ax, ay, az = jnp.abs(x), jnp.abs(y), jnp.abs(z)
    m_front = (z > 0) & (z > ax) & (z > ay)
    m_back = (z < 0) & (-z > ax) & (-z > ay)
    m_right = (x > 0) & (x > az) & (x > ay)
    m_left = (x < 0) & (-x > az) & (-x > ay)

    f = S / 2.0

    def pix(px_num, py_num, zden):
        z_safe = jnp.where(jnp.abs(zden) > 1e-9, zden, 1.0)
        px = f * px_num / z_safe + f
        py = f * py_num / z_safe + f
        iu = jnp.clip(jnp.floor(px), 0, S - 1).astype(jnp.int32)
        iv = jnp.clip(jnp.floor(py), 0, S - 1).astype(jnp.int32)
        return iu * S + iv

    lin_f = pix(x, y, z)
    lin_b = pix(x, -y, z)
    lin_r = pix(-z, y, x)
    lin_l = pix(z, y, -x)

    pid = pl.program_id(0)
    row = jax.lax.broadcasted_iota(jnp.int32, lat.shape, 0)
    col = jax.lax.broadcasted_iota(jnp.int32, lat.shape, 1)
    gidx = (pid * _BLK + row) * 128 + col
    valid = gidx < n_ref[0]

    sent = jnp.int32(SENT)
    lin = jnp.where(valid & m_front, lin_f,
          jnp.where(valid & m_back, SS + lin_b,
          jnp.where(valid & m_left, 2 * SS + lin_l,
          jnp.where(valid & m_right, 3 * SS + lin_r, sent))))
    idx_ref[...] = lin


def _compute_lin_idx(lat, lon, alt, cam, n):
    return pl.pallas_call(
        _transform_body,
        grid=(_N_GRID,),
        in_specs=[
            pl.BlockSpec((_BLK, 128), lambda i: (i, 0)),
            pl.BlockSpec((_BLK, 128), lambda i: (i, 0)),
            pl.BlockSpec((_BLK, 128), lambda i: (i, 0)),
            pl.BlockSpec((1, 8), lambda i: (0, 0)),
            pl.BlockSpec(memory_space=pltpu.SMEM),
        ],
        out_specs=pl.BlockSpec((_BLK, 128), lambda i: (i, 0)),
        out_shape=jax.ShapeDtypeStruct((_ROWS, 128), jnp.int32),
    )(lat, lon, alt, cam, n)


def kernel(points, cam_params):
    npts = points.shape[0]
    npad = _ROWS * 128
    lat = jnp.zeros((npad,), jnp.float32).at[:npts].set(points[:, 0]).reshape(_ROWS, 128)
    lon = jnp.zeros((npad,), jnp.float32).at[:npts].set(points[:, 1]).reshape(_ROWS, 128)
    alt = jnp.zeros((npad,), jnp.float32).at[:npts].set(points[:, 2]).reshape(_ROWS, 128)
    inten = jnp.zeros((npad,), jnp.float32).at[:npts].set(points[:, 3])
    cam = jnp.zeros((1, 8), jnp.float32).at[0, :7].set(cam_params)
    n = jnp.full((1,), npts, dtype=jnp.int32)

    lin = _compute_lin_idx(lat, lon, alt, cam, n).reshape(-1)

    flat = jnp.zeros((4 * SS + 1,), jnp.float32).at[lin].set(inten)
    front = flat[0:SS].reshape(S, S)
    back = flat[SS:2 * SS].reshape(S, S)
    left = flat[2 * SS:3 * SS].reshape(S, S)
    right = flat[3 * SS:4 * SS].reshape(S, S)
    return front, back, left, right


# speedup vs baseline: 1.2135x; 1.2135x over previous
import functools

import jax
import jax.numpy as jnp
from jax.experimental import pallas as pl
from jax.experimental.pallas import tpu as pltpu

S = 2048
SS = S * S
SENT = 4 * SS

_ROWS = 8192
_BLK = 1024
_N_GRID = _ROWS // _BLK


def _transform_body(lat_ref, lon_ref, alt_ref, cam_ref, n_ref, idx_ref):
    lat = lat_ref[...]
    lon = lon_ref[...]
    alt = alt_ref[...]

    lat0 = cam_ref[0, 0]
    lon0 = cam_ref[0, 1]
    alt0 = cam_ref[0, 2]
    qs = cam_ref[0, 3]
    qx = cam_ref[0, 4]
    qy = cam_ref[0, 5]
    qz = cam_ref[0, 6]

    a = 6378137.0
    e2 = 6.69437999014e-3

    latr = jnp.deg2rad(lat)
    lonr = jnp.deg2rad(lon)
    s = jnp.sin(latr)
    c = jnp.cos(latr)
    Nv = a * jax.lax.rsqrt(1.0 - e2 * s * s)
    X = (Nv + alt) * c * jnp.cos(lonr)
    Y = (Nv + alt) * c * jnp.sin(lonr)
    Z = (Nv * (1.0 - e2) + alt) * s

    latr0 = jnp.deg2rad(lat0)
    lonr0 = jnp.deg2rad(lon0)
    s0 = jnp.sin(latr0)
    c0 = jnp.cos(latr0)
    Nv0 = a * jax.lax.rsqrt(1.0 - e2 * s0 * s0)
    x0 = (Nv0 + alt0) * c0 * jnp.cos(lonr0)
    y0 = (Nv0 + alt0) * c0 * jnp.sin(lonr0)
    z0 = (Nv0 * (1.0 - e2) + alt0) * s0

    dx, dy, dz = X - x0, Y - y0, Z - z0
    sl, cl = jnp.sin(latr0), jnp.cos(latr0)
    so, co = jnp.sin(lonr0), jnp.cos(lonr0)
    e = -so * dx + co * dy
    n = -sl * co * dx - sl * so * dy + cl * dz
    u = cl * co * dx + cl * so * dy + sl * dz

    nqs = -qs
    nrm = jnp.sqrt(nqs * nqs + qx * qx + qy * qy + qz * qz) + 1e-12
    rqs, rqx, rqy, rqz = nqs / nrm, qx / nrm, qy / nrm, qz / nrm
    x = ((1 - 2 * (rqy * rqy + rqz * rqz)) * e
         + 2 * (rqx * rqy - rqz * rqs) * n
         + 2 * (rqx * rqz + rqy * rqs) * u)
    y = (2 * (rqx * rqy + rqz * rqs) * e
         + (1 - 2 * (rqx * rqx + rqz * rqz)) * n
         + 2 * (rqy * rqz - rqx * rqs) * u)
    z = (2 * (rqx * rqz - rqy * rqs) * e
         + 2 * (rqy * rqz + rqx * rqs) * n
         + (1 - 2 * (rqx * rqx + rqy * rqy)) * u)

    ax, ay, az = jnp.abs(x), jnp.abs(y), jnp.abs(z)
    m_front = (z > 0) & (z > ax) & (z > ay)
    m_back = (z < 0) & (-z > ax) & (-z > ay)
    m_right = (x > 0) & (x > az) & (x > ay)
    m_left = (x < 0) & (-x > az) & (-x > ay)

    f = S / 2.0

    def pix(px_num, py_num, zden):
        z_safe = jnp.where(jnp.abs(zden) > 1e-9, zden, 1.0)
        px = f * px_num / z_safe + f
        py = f * py_num / z_safe + f
        iu = jnp.clip(jnp.floor(px), 0, S - 1).astype(jnp.int32)
        iv = jnp.clip(jnp.floor(py), 0, S - 1).astype(jnp.int32)
        return iu * S + iv

    lin_f = pix(x, y, z)
    lin_b = pix(x, -y, z)
    lin_r = pix(-z, y, x)
    lin_l = pix(z, y, -x)

    pid = pl.program_id(0)
    row = jax.lax.broadcasted_iota(jnp.int32, lat.shape, 0)
    col = jax.lax.broadcasted_iota(jnp.int32, lat.shape, 1)
    gidx = (pid * _BLK + row) * 128 + col
    valid = gidx < n_ref[0]

    sent = jnp.int32(SENT)
    lin = jnp.where(valid & m_front, lin_f,
          jnp.where(valid & m_back, SS + lin_b,
          jnp.where(valid & m_left, 2 * SS + lin_l,
          jnp.where(valid & m_right, 3 * SS + lin_r, sent))))
    idx_ref[...] = lin


def _compute_lin_idx(lat, lon, alt, cam, n):
    return pl.pallas_call(
        _transform_body,
        grid=(_N_GRID,),
        in_specs=[
            pl.BlockSpec((_BLK, 128), lambda i: (i, 0)),
            pl.BlockSpec((_BLK, 128), lambda i: (i, 0)),
            pl.BlockSpec((_BLK, 128), lambda i: (i, 0)),
            pl.BlockSpec((1, 8), lambda i: (0, 0)),
            pl.BlockSpec(memory_space=pltpu.SMEM),
        ],
        out_specs=pl.BlockSpec((_BLK, 128), lambda i: (i, 0)),
        out_shape=jax.ShapeDtypeStruct((_ROWS, 128), jnp.int32),
    )(lat, lon, alt, cam, n)


def kernel(points, cam_params):
    npts = points.shape[0]
    npad = _ROWS * 128
    lat = jnp.zeros((npad,), jnp.float32).at[:npts].set(points[:, 0]).reshape(_ROWS, 128)
    lon = jnp.zeros((npad,), jnp.float32).at[:npts].set(points[:, 1]).reshape(_ROWS, 128)
    alt = jnp.zeros((npad,), jnp.float32).at[:npts].set(points[:, 2]).reshape(_ROWS, 128)
    inten = jnp.zeros((npad,), jnp.float32).at[:npts].set(points[:, 3])
    cam = jnp.zeros((1, 8), jnp.float32).at[0, :7].set(cam_params)
    n = jnp.full((1,), npts, dtype=jnp.int32)

    lin = _compute_lin_idx(lat, lon, alt, cam, n).reshape(-1)

    outs = []
    for face in range(4):
        base = face * SS
        on_face = (lin >= base) & (lin < base + SS)
        rel = lin - base
        iu = jnp.where(on_face, rel >> 11, S)
        iv = jnp.where(on_face, rel & (S - 1), S)
        img = jnp.zeros((S + 1, S + 1), jnp.float32).at[iu, iv].set(inten)
        outs.append(img[:S, :S])
    return outs[0], outs[1], outs[2], outs[3]
